# baseline (speedup 1.0000x reference)
import functools

import jax
import jax.numpy as jnp
from jax import lax
from jax.experimental import pallas as pl
from jax.experimental.pallas import tpu as pltpu
from jax.experimental.pallas import tpu_sc as plsc

D = 64
NW = 32
IDX_MINOR = 128
K = 4
CHUNK = K * IDX_MINOR
NBUF = 2


def kernel(x, W):
    B, S = x.shape
    total = B * S
    per_w = total // NW
    n_steps = per_w // IDX_MINOR
    n_chunks = per_w // CHUNK
    n_pairs = n_chunks // NBUF
    idx3 = x.reshape(NW, n_steps, IDX_MINOR)

    mesh = plsc.VectorSubcoreMesh(core_axis_name="c", subcore_axis_name="s")

    @functools.partial(
        pl.kernel,
        out_type=jax.ShapeDtypeStruct((total, D), jnp.float32),
        mesh=mesh,
        compiler_params=pltpu.CompilerParams(use_tc_tiling_on_sc=False),
        scratch_types=[
            pltpu.VMEM((n_steps, IDX_MINOR), jnp.int32),
            pltpu.VMEM((NBUF, CHUNK, D), jnp.float32),
            pltpu.SemaphoreType.DMA,
            pltpu.SemaphoreType.DMA,
        ],
    )
    def run(idx_hbm, table_hbm, out_hbm, idx_v, rows_v, gsem0, gsem1):
        wid = lax.axis_index("s") * 2 + lax.axis_index("c")
        base = wid * per_w
        pltpu.sync_copy(idx_hbm.at[wid], idx_v)
        gsems = (gsem0, gsem1)

        def fire(g, b):
            for k in range(K):
                pltpu.async_copy(
                    table_hbm.at[idx_v.at[g * K + k]],
                    rows_v.at[b].at[pl.ds(k * IDX_MINOR, IDX_MINOR)],
                    gsems[b],
                )

        def drain(b):
            pltpu.make_async_copy(
                table_hbm.at[pl.ds(0, CHUNK)], rows_v.at[b], gsems[b]
            ).wait()

        fire(0, 0)
        fire(1, 1)

        def body(p, _):
            for b in range(NBUF):
                g = p * NBUF + b
                drain(b)
                pltpu.sync_copy(
                    rows_v.at[b], out_hbm.at[pl.ds(base + g * CHUNK, CHUNK)]
                )

                @pl.when(p != n_pairs - 1)
                def _():
                    fire(g + NBUF, b)
            return _

        lax.fori_loop(0, n_pairs, body, None)

    out = run(idx3, W)
    return out.reshape(B, S, D)


# speedup vs baseline: 1.0016x; 1.0016x over previous
import functools

import jax
import jax.numpy as jnp
from jax import lax
from jax.experimental import pallas as pl
from jax.experimental.pallas import tpu as pltpu
from jax.experimental.pallas import tpu_sc as plsc

D = 64
NW = 32
NB = 2
NG = 5
GSZ = 40
NBUF = 2


def kernel(x, W):
    B, S = x.shape
    rows_per_w = B // NW
    n_chunks = rows_per_w // NB
    n_pairs = n_chunks // NBUF

    mesh = plsc.VectorSubcoreMesh(core_axis_name="c", subcore_axis_name="s")

    @functools.partial(
        pl.kernel,
        out_type=jax.ShapeDtypeStruct((B, S, D), jnp.float32),
        mesh=mesh,
        compiler_params=pltpu.CompilerParams(use_tc_tiling_on_sc=False),
        scratch_types=[
            pltpu.VMEM((rows_per_w, S), jnp.int32),
            pltpu.VMEM((NBUF, NB, S, D), jnp.float32),
            pltpu.SemaphoreType.DMA,
            pltpu.SemaphoreType.DMA,
        ],
    )
    def run(idx_hbm, table_hbm, out_hbm, idx_v, rows_v, gsem0, gsem1):
        wid = lax.axis_index("s") * 2 + lax.axis_index("c")
        base = wid * rows_per_w
        pltpu.sync_copy(idx_hbm.at[pl.ds(base, rows_per_w)], idx_v)
        gsems = (gsem0, gsem1)

        def fire(g, b):
            for r in range(NB):
                for h in range(NG):
                    pltpu.async_copy(
                        table_hbm.at[idx_v.at[g * NB + r, pl.ds(h * GSZ, GSZ)]],
                        rows_v.at[b, r, pl.ds(h * GSZ, GSZ)],
                        gsems[b],
                    )

        def drain(b):
            pltpu.make_async_copy(
                out_hbm.at[pl.ds(0, NB)], rows_v.at[b], gsems[b]
            ).wait()

        fire(0, 0)
        fire(1, 1)

        def body(p, _):
            for b in range(NBUF):
                g = p * NBUF + b
                drain(b)
                pltpu.sync_copy(
                    rows_v.at[b], out_hbm.at[pl.ds(base + g * NB, NB)]
                )

                @pl.when(p != n_pairs - 1)
                def _():
                    fire(g + NBUF, b)
            return _

        lax.fori_loop(0, n_pairs, body, None)

    return run(x, W)
